# baseline (speedup 1.0000x reference)
import functools
import math

import jax
import jax.numpy as jnp
from jax import lax
from jax.experimental import pallas as pl
from jax.experimental.pallas import tpu as pltpu
from jax.experimental.pallas import tpu_sc as plsc

_ALPHA = 0.001
_SPARSITY = 0.05

_NC = 2
_NS = 16
_NW = _NC * _NS
_LANES = 16
_NBINS = 1 << 15
_SUB = 16384
_BPT = _NBINS // _NS


def _normalize_kernel(pre_ref, post_ref, perm_ref, out_ref):
    prod = jax.lax.dot_general(
        pre_ref[...], post_ref[...], (((0,), (0,)), ((), ())),
        preferred_element_type=jnp.float32)
    p = perm_ref[...] + _ALPHA * prod
    s = jnp.sum(p, axis=0, keepdims=True)
    out_ref[...] = p / s


def _sc_find_bin(tab_hbm, piece_ref, k):
    npieces = _NBINS // 2048

    def piece_body(i, carry):
        acc, b1, krem = carry
        p = npieces - 1 - i
        pltpu.sync_copy(tab_hbm.at[0, pl.ds(p * 2048, 2048)], piece_ref.at[0])
        pltpu.sync_copy(tab_hbm.at[1, pl.ds(p * 2048, 2048)], piece_ref.at[1])

        def vreg_body(i2, carry2):
            acc2, b12, krem2 = carry2
            j2 = 127 - i2
            v = piece_ref[0, pl.ds(j2 * _LANES, _LANES)] + \
                piece_ref[1, pl.ds(j2 * _LANES, _LANES)]
            s = jnp.sum(v)
            pre = plsc.cumsum(v)
            hit = (acc2 < k) & (acc2 + s >= k)
            suffix = acc2 + (s - pre + v)
            lanes = lax.iota(jnp.int32, _LANES)
            cand = jnp.where(suffix >= k, lanes, jnp.full((_LANES,), -1, jnp.int32))
            lmax = jnp.max(cand)
            pre_l = jnp.sum(jnp.where(lanes == lmax, pre, jnp.zeros((_LANES,), jnp.int32)))
            base_bin = p * 2048 + j2 * _LANES
            b_new = jnp.where(hit, base_bin + lmax, b12)
            krem_new = jnp.where(hit, k - (acc2 + s - pre_l), krem2)
            return (acc2 + s, b_new, krem_new)

        return lax.fori_loop(0, 128, vreg_body, (acc, b1, krem))

    _, b1, krem = lax.fori_loop(
        0, npieces, piece_body,
        (jnp.int32(0), jnp.int32(0), jnp.int32(0)))
    return b1, krem


def _sc_hist_body(pf_hbm, chunk, data_ref, table_ref, level, b1):
    cid = lax.axis_index("c")
    sid = lax.axis_index("s")
    wid = cid * _NS + sid
    base = wid * chunk

    def zbody(i, _):
        table_ref[pl.ds(i * _LANES, _LANES)] = jnp.zeros((_LANES,), jnp.int32)
        return 0
    lax.fori_loop(0, _NBINS // _LANES, zbody, 0)

    ones = jnp.ones((_LANES,), jnp.int32)

    def sub_body(s, _):
        pltpu.sync_copy(pf_hbm.at[pl.ds(base + s * _SUB, _SUB)], data_ref)

        def vbody(i, _):
            u = data_ref[pl.ds(i * _LANES, _LANES)]
            if level == 0:
                b = jax.lax.shift_right_logical(u, 15)
                plsc.addupdate_scatter(table_ref, [b], ones)
            else:
                hi = jax.lax.shift_right_logical(u, 15)
                b = jnp.bitwise_and(u, jnp.int32(_NBINS - 1))
                plsc.addupdate_scatter(table_ref, [b], ones,
                                       mask=(hi == b1))
            return 0
        lax.fori_loop(0, _SUB // _LANES, vbody, 0)
        return 0
    lax.fori_loop(0, chunk // _SUB, sub_body, 0)


def _sc_merge_and_emit(out_hbm, table_ref, shared_ref, slice_ref):
    cid = lax.axis_index("c")
    sid = lax.axis_index("s")
    pltpu.sync_copy(table_ref, shared_ref.at[sid])
    plsc.subcore_barrier()
    pltpu.sync_copy(shared_ref.at[:, pl.ds(sid * _BPT, _BPT)], slice_ref)

    def rbody(i, _):
        acc = jnp.zeros((_LANES,), jnp.int32)
        for r in range(_NS):
            acc = acc + slice_ref[r, pl.ds(i * _LANES, _LANES)]
        table_ref[pl.ds(i * _LANES, _LANES)] = acc
        return 0
    lax.fori_loop(0, _BPT // _LANES, rbody, 0)
    pltpu.sync_copy(table_ref.at[pl.ds(0, _BPT)],
                    out_hbm.at[cid, pl.ds(sid * _BPT, _BPT)])


def _sc_mesh():
    return plsc.VectorSubcoreMesh(
        core_axis_name="c", subcore_axis_name="s",
        num_cores=_NC, num_subcores=_NS)


def _make_sc_hist1(n):
    chunk = n // _NW
    mesh = _sc_mesh()

    @functools.partial(
        pl.kernel,
        out_type=jax.ShapeDtypeStruct((_NC, _NBINS), jnp.int32),
        mesh=mesh,
        compiler_params=pltpu.CompilerParams(needs_layout_passes=False),
        scratch_types=[
            pltpu.VMEM((_SUB,), jnp.int32),
            pltpu.VMEM((_NBINS,), jnp.int32),
            pltpu.VMEM_SHARED((_NS, _NBINS), jnp.int32),
            pltpu.VMEM((_NS, _BPT), jnp.int32),
        ],
    )
    def hist1(pf_hbm, out_hbm, data_ref, table_ref, shared_ref, slice_ref):
        _sc_hist_body(pf_hbm, chunk, data_ref, table_ref, 0, None)
        _sc_merge_and_emit(out_hbm, table_ref, shared_ref, slice_ref)

    return hist1


def _make_sc_hist2(n, k):
    chunk = n // _NW
    mesh = _sc_mesh()

    @functools.partial(
        pl.kernel,
        out_type=jax.ShapeDtypeStruct((_NC, _NBINS), jnp.int32),
        mesh=mesh,
        compiler_params=pltpu.CompilerParams(needs_layout_passes=False),
        scratch_types=[
            pltpu.VMEM((_SUB,), jnp.int32),
            pltpu.VMEM((_NBINS,), jnp.int32),
            pltpu.VMEM_SHARED((_NS, _NBINS), jnp.int32),
            pltpu.VMEM((_NS, _BPT), jnp.int32),
            pltpu.VMEM((2, 2048), jnp.int32),
        ],
    )
    def hist2(pf_hbm, t1_hbm, out_hbm, data_ref, table_ref, shared_ref,
              slice_ref, piece_ref):
        b1, _ = _sc_find_bin(t1_hbm, piece_ref, k)
        _sc_hist_body(pf_hbm, chunk, data_ref, table_ref, 1, b1)
        _sc_merge_and_emit(out_hbm, table_ref, shared_ref, slice_ref)

    return hist2


def _derive_kernel(k, t1_ref, t2_ref, t_ref, need_ref, c2_ref):
    bins = jax.lax.broadcasted_iota(jnp.int32, (1, _NBINS), 1)

    def find(m, kk):
        def bit_body(i, b):
            trial = jnp.bitwise_or(b, jax.lax.shift_left(jnp.int32(1), jnp.int32(14) - i))
            s = jnp.sum(jnp.where(bins >= trial, m, 0))
            return jnp.where(s >= kk, trial, b)
        return jax.lax.fori_loop(0, 15, bit_body, jnp.int32(0))

    m1 = jnp.sum(t1_ref[...], axis=0, keepdims=True)
    b1 = find(m1, k)
    krem = k - jnp.sum(jnp.where(bins > b1, m1, 0))
    m2 = jnp.sum(t2_ref[...], axis=0, keepdims=True)
    b2 = find(m2, krem)
    s2_excl = jnp.sum(jnp.where(bins > b2, m2, 0))
    c2 = jnp.sum(jnp.where(bins == b2, m2, 0))
    t_ref[0, 0] = jnp.bitwise_or(jax.lax.shift_left(b1, 15), b2)
    need_ref[0, 0] = krem - s2_excl
    c2_ref[0, 0] = c2


def _tie_kernel(n_chunks, pn_ref, t_ref, need_ref, c_ref):
    n_rows, n_cols = pn_ref.shape
    r = n_rows // n_chunks
    t = t_ref[0, 0]
    need = need_ref[0, 0]

    def count_eq_lt(cut):
        def body(ci, acc):
            sl = pl.ds(pl.multiple_of(ci * r, r), r)
            u = jax.lax.bitcast_convert_type(pn_ref[sl, :], jnp.int32)
            rows = jax.lax.broadcasted_iota(jnp.int32, (r, n_cols), 0)
            cols = jax.lax.broadcasted_iota(jnp.int32, (r, n_cols), 1)
            flat = (rows + ci * r) * n_cols + cols
            return acc + jnp.sum(((u == t) & (flat < cut)).astype(jnp.int32))
        return jax.lax.fori_loop(0, n_chunks, body, jnp.int32(0))

    def c_body(i, c):
        trial = jnp.bitwise_or(c, jax.lax.shift_left(jnp.int32(1), jnp.int32(23) - i))
        return jnp.where(count_eq_lt(trial) <= need, trial, c)

    c_ref[0, 0] = jax.lax.fori_loop(0, 24, c_body, jnp.int32(0))


def _mask_kernel(row_block, t_ref, c_ref, pn_ref, mask_ref):
    u = jax.lax.bitcast_convert_type(pn_ref[...], jnp.int32)
    t = t_ref[0, 0]
    cut = c_ref[0, 0]
    n_cols = u.shape[1]
    rows = jax.lax.broadcasted_iota(jnp.int32, u.shape, 0)
    cols = jax.lax.broadcasted_iota(jnp.int32, u.shape, 1)
    flat = (rows + pl.program_id(0) * row_block) * n_cols + cols
    mask_ref[...] = ((u > t) | ((u == t) & (flat < cut))).astype(jnp.int32)


def _update_one(pre, post, perm, col_block=512, row_block=256, n_chunks=16):
    n_pre, n_post = perm.shape
    b = pre.shape[0]
    n = n_pre * n_post
    k = math.ceil(n * _SPARSITY)
    nc = n_post // col_block
    nr = n_pre // row_block

    pn = pl.pallas_call(
        _normalize_kernel,
        grid=(nc,),
        in_specs=[
            pl.BlockSpec((b, n_pre), lambda j: (0, 0)),
            pl.BlockSpec((b, col_block), lambda j: (0, j)),
            pl.BlockSpec((n_pre, col_block), lambda j: (0, j)),
        ],
        out_specs=pl.BlockSpec((n_pre, col_block), lambda j: (0, j)),
        out_shape=jax.ShapeDtypeStruct((n_pre, n_post), jnp.float32),
    )(pre, post, perm)

    pf = jax.lax.bitcast_convert_type(pn.reshape(-1), jnp.int32)
    t1 = _make_sc_hist1(n)(pf)
    t2 = _make_sc_hist2(n, k)(pf, t1)

    t, need, c2 = pl.pallas_call(
        functools.partial(_derive_kernel, k),
        out_specs=[pl.BlockSpec(memory_space=pltpu.SMEM)] * 3,
        out_shape=[jax.ShapeDtypeStruct((1, 1), jnp.int32)] * 3,
    )(t1, t2)

    def all_ties(ops):
        return jnp.full((1, 1), 1 << 24, jnp.int32)

    def exact_ties(ops):
        pn_, t_, need_ = ops
        return pl.pallas_call(
            functools.partial(_tie_kernel, n_chunks),
            in_specs=[
                pl.BlockSpec((n_pre, n_post), lambda: (0, 0)),
                pl.BlockSpec(memory_space=pltpu.SMEM),
                pl.BlockSpec(memory_space=pltpu.SMEM),
            ],
            out_specs=pl.BlockSpec(memory_space=pltpu.SMEM),
            out_shape=jax.ShapeDtypeStruct((1, 1), jnp.int32),
        )(pn_, t_, need_)

    cut = jax.lax.cond(c2[0, 0] == need[0, 0], all_ties, exact_ties,
                       (pn, t, need))

    mask = pl.pallas_call(
        functools.partial(_mask_kernel, row_block),
        grid=(nr,),
        in_specs=[
            pl.BlockSpec(memory_space=pltpu.SMEM),
            pl.BlockSpec(memory_space=pltpu.SMEM),
            pl.BlockSpec((row_block, n_post), lambda i: (i, 0)),
        ],
        out_specs=pl.BlockSpec((row_block, n_post), lambda i: (i, 0)),
        out_shape=jax.ShapeDtypeStruct((n_pre, n_post), jnp.int32),
    )(t, cut, pn)
    return pn, mask


def kernel(x, h, y, perm_xy, perm_xh, perm_hy):
    pn_xy, w_xy = _update_one(x, y, perm_xy)
    pn_xh, w_xh = _update_one(x, h, perm_xh)
    pn_hy, w_hy = _update_one(h, y, perm_hy)
    return (w_xy, w_xh, w_hy, pn_xy, pn_xh, pn_hy)


# speedup vs baseline: 1.3361x; 1.3361x over previous
import functools
import math

import jax
import jax.numpy as jnp
from jax import lax
from jax.experimental import pallas as pl
from jax.experimental.pallas import tpu as pltpu
from jax.experimental.pallas import tpu_sc as plsc

_ALPHA = 0.001
_SPARSITY = 0.05

_NC = 2
_NS = 16
_NW = _NC * _NS
_LANES = 16
_NBINS = 1 << 15
_SUB = 16384
_BPT = _NBINS // _NS
_UNROLL = 8


def _normalize_kernel(pre_ref, post_ref, perm_ref, out_ref):
    prod = jax.lax.dot_general(
        pre_ref[...], post_ref[...], (((0,), (0,)), ((), ())),
        preferred_element_type=jnp.float32)
    p = perm_ref[...] + _ALPHA * prod
    s = jnp.sum(p, axis=0, keepdims=True)
    out_ref[...] = p / s


def _sc_hist_body(pf_hbm, chunk, d_refs, sems, table_ref, level, b1):
    cid = lax.axis_index("c")
    sid = lax.axis_index("s")
    wid = cid * _NS + sid
    base = wid * chunk
    nsub = chunk // _SUB

    def zbody(i, _):
        for j in range(_UNROLL):
            table_ref[pl.ds((i * _UNROLL + j) * _LANES, _LANES)] = (
                jnp.zeros((_LANES,), jnp.int32))
        return 0
    lax.fori_loop(0, _NBINS // _LANES // _UNROLL, zbody, 0)

    ones = jnp.ones((_LANES,), jnp.int32)

    def start(s, bref, sem):
        off = base + jnp.minimum(s, nsub - 1) * _SUB
        pltpu.async_copy(pf_hbm.at[pl.ds(off, _SUB)], bref, sem)

    def wait(bref, sem):
        pltpu.make_async_copy(pf_hbm.at[pl.ds(base, _SUB)], bref, sem).wait()

    def process(bref):
        def vbody(i, _):
            for j in range(_UNROLL):
                u = bref[pl.ds((i * _UNROLL + j) * _LANES, _LANES)]
                if level == 0:
                    b = jax.lax.shift_right_logical(u, 15)
                    plsc.addupdate_scatter(table_ref, [b], ones)
                else:
                    hi = jax.lax.shift_right_logical(u, 15)
                    b = jnp.bitwise_and(u, jnp.int32(_NBINS - 1))
                    plsc.addupdate_scatter(table_ref, [b], ones,
                                           mask=(hi == b1))
            return 0
        lax.fori_loop(0, _SUB // _LANES // _UNROLL, vbody, 0)

    start(0, d_refs[0], sems[0])
    start(1, d_refs[1], sems[1])

    def pair_body(p, _):
        s = p * 2
        wait(d_refs[0], sems[0])
        process(d_refs[0])
        start(s + 2, d_refs[0], sems[0])
        wait(d_refs[1], sems[1])
        process(d_refs[1])
        start(s + 3, d_refs[1], sems[1])
        return 0
    lax.fori_loop(0, nsub // 2, pair_body, 0)
    wait(d_refs[0], sems[0])
    wait(d_refs[1], sems[1])


_SLICE = 512


def _sc_merge_and_emit(out_hbm, table_ref, shared_ref, slice_ref):
    cid = lax.axis_index("c")
    sid = lax.axis_index("s")
    pltpu.sync_copy(table_ref, shared_ref.at[sid])
    plsc.subcore_barrier()
    for piece in range(_BPT // _SLICE):
        pltpu.sync_copy(
            shared_ref.at[:, pl.ds(sid * _BPT + piece * _SLICE, _SLICE)],
            slice_ref)

        def rbody(i, _):
            acc = jnp.zeros((_LANES,), jnp.int32)
            for r in range(_NS):
                acc = acc + slice_ref[r, pl.ds(i * _LANES, _LANES)]
            table_ref[pl.ds(piece * _SLICE + i * _LANES, _LANES)] = acc
            return 0
        lax.fori_loop(0, _SLICE // _LANES, rbody, 0)
    pltpu.sync_copy(table_ref.at[pl.ds(0, _BPT)],
                    out_hbm.at[cid, pl.ds(sid * _BPT, _BPT)])


def _sc_mesh():
    return plsc.VectorSubcoreMesh(
        core_axis_name="c", subcore_axis_name="s",
        num_cores=_NC, num_subcores=_NS)


def _make_sc_hist1(n):
    chunk = n // _NW

    @functools.partial(
        pl.kernel,
        out_type=jax.ShapeDtypeStruct((_NC, _NBINS), jnp.int32),
        mesh=_sc_mesh(),
        compiler_params=pltpu.CompilerParams(needs_layout_passes=False),
        scratch_types=[
            pltpu.VMEM((_SUB,), jnp.int32),
            pltpu.VMEM((_SUB,), jnp.int32),
            pltpu.VMEM((_NBINS,), jnp.int32),
            pltpu.VMEM_SHARED((_NS, _NBINS), jnp.int32),
            pltpu.VMEM((_NS, _SLICE), jnp.int32),
            pltpu.SemaphoreType.DMA,
            pltpu.SemaphoreType.DMA,
        ],
    )
    def hist1(pf_hbm, out_hbm, d0, d1, table_ref, shared_ref, slice_ref,
              sem0, sem1):
        _sc_hist_body(pf_hbm, chunk, (d0, d1), (sem0, sem1), table_ref,
                      0, None)
        _sc_merge_and_emit(out_hbm, table_ref, shared_ref, slice_ref)

    return hist1


def _make_sc_hist2(n):
    chunk = n // _NW

    @functools.partial(
        pl.kernel,
        out_type=jax.ShapeDtypeStruct((_NC, _NBINS), jnp.int32),
        mesh=_sc_mesh(),
        compiler_params=pltpu.CompilerParams(needs_layout_passes=False),
        scratch_types=[
            pltpu.VMEM((_SUB,), jnp.int32),
            pltpu.VMEM((_SUB,), jnp.int32),
            pltpu.VMEM((_NBINS,), jnp.int32),
            pltpu.VMEM_SHARED((_NS, _NBINS), jnp.int32),
            pltpu.VMEM((_NS, _SLICE), jnp.int32),
            pltpu.VMEM((_LANES,), jnp.int32),
            pltpu.SemaphoreType.DMA,
            pltpu.SemaphoreType.DMA,
        ],
    )
    def hist2(pf_hbm, b1_hbm, out_hbm, d0, d1, table_ref, shared_ref,
              slice_ref, b1_ref, sem0, sem1):
        pltpu.sync_copy(b1_hbm.at[0, pl.ds(0, _LANES)], b1_ref)
        v = b1_ref[...]
        lanes = lax.iota(jnp.int32, _LANES)
        b1 = jnp.sum(jnp.where(lanes == 0, v, jnp.zeros((_LANES,), jnp.int32)))
        _sc_hist_body(pf_hbm, chunk, (d0, d1), (sem0, sem1), table_ref,
                      1, b1)
        _sc_merge_and_emit(out_hbm, table_ref, shared_ref, slice_ref)

    return hist2


def _suffix_find(m, bins, kk):
    def bit_body(i, b):
        trial = jnp.bitwise_or(b, jax.lax.shift_left(jnp.int32(1), jnp.int32(14) - i))
        s = jnp.sum(jnp.where(bins >= trial, m, 0))
        return jnp.where(s >= kk, trial, b)
    return jax.lax.fori_loop(0, 15, bit_body, jnp.int32(0))


def _find_b1_kernel(k, t1_ref, out_ref):
    bins = jax.lax.broadcasted_iota(jnp.int32, (1, _NBINS), 1)
    m1 = jnp.sum(t1_ref[...], axis=0, keepdims=True)
    b1 = _suffix_find(m1, bins, k)
    lanes = jax.lax.broadcasted_iota(jnp.int32, (1, 128), 1)
    out_ref[...] = jnp.where(lanes == 0, b1, 0)


def _derive_kernel(k, t1_ref, t2_ref, t_ref, need_ref, c2_ref):
    bins = jax.lax.broadcasted_iota(jnp.int32, (1, _NBINS), 1)
    m1 = jnp.sum(t1_ref[...], axis=0, keepdims=True)
    b1 = _suffix_find(m1, bins, k)
    krem = k - jnp.sum(jnp.where(bins > b1, m1, 0))
    m2 = jnp.sum(t2_ref[...], axis=0, keepdims=True)
    b2 = _suffix_find(m2, bins, krem)
    s2_excl = jnp.sum(jnp.where(bins > b2, m2, 0))
    c2 = jnp.sum(jnp.where(bins == b2, m2, 0))
    t_ref[0, 0] = jnp.bitwise_or(jax.lax.shift_left(b1, 15), b2)
    need_ref[0, 0] = krem - s2_excl
    c2_ref[0, 0] = c2


def _tie_kernel(n_chunks, pn_ref, t_ref, need_ref, c_ref):
    n_rows, n_cols = pn_ref.shape
    r = n_rows // n_chunks
    t = t_ref[0, 0]
    need = need_ref[0, 0]

    def count_eq_lt(cut):
        def body(ci, acc):
            sl = pl.ds(pl.multiple_of(ci * r, r), r)
            u = jax.lax.bitcast_convert_type(pn_ref[sl, :], jnp.int32)
            rows = jax.lax.broadcasted_iota(jnp.int32, (r, n_cols), 0)
            cols = jax.lax.broadcasted_iota(jnp.int32, (r, n_cols), 1)
            flat = (rows + ci * r) * n_cols + cols
            return acc + jnp.sum(((u == t) & (flat < cut)).astype(jnp.int32))
        return jax.lax.fori_loop(0, n_chunks, body, jnp.int32(0))

    def c_body(i, c):
        trial = jnp.bitwise_or(c, jax.lax.shift_left(jnp.int32(1), jnp.int32(23) - i))
        return jnp.where(count_eq_lt(trial) <= need, trial, c)

    c_ref[0, 0] = jax.lax.fori_loop(0, 24, c_body, jnp.int32(0))


def _mask_kernel(row_block, t_ref, c_ref, pn_ref, mask_ref):
    u = jax.lax.bitcast_convert_type(pn_ref[...], jnp.int32)
    t = t_ref[0, 0]
    cut = c_ref[0, 0]
    n_cols = u.shape[1]
    rows = jax.lax.broadcasted_iota(jnp.int32, u.shape, 0)
    cols = jax.lax.broadcasted_iota(jnp.int32, u.shape, 1)
    flat = (rows + pl.program_id(0) * row_block) * n_cols + cols
    mask_ref[...] = ((u > t) | ((u == t) & (flat < cut))).astype(jnp.int32)


def _normalize(pre, post, perm, col_block=512):
    n_pre, n_post = perm.shape
    b = pre.shape[0]
    nc = n_post // col_block
    return pl.pallas_call(
        _normalize_kernel,
        grid=(nc,),
        in_specs=[
            pl.BlockSpec((b, n_pre), lambda j: (0, 0)),
            pl.BlockSpec((b, col_block), lambda j: (0, j)),
            pl.BlockSpec((n_pre, col_block), lambda j: (0, j)),
        ],
        out_specs=pl.BlockSpec((n_pre, col_block), lambda j: (0, j)),
        out_shape=jax.ShapeDtypeStruct((n_pre, n_post), jnp.float32),
    )(pre, post, perm)


def _find_b1(k, t1):
    return pl.pallas_call(
        functools.partial(_find_b1_kernel, k),
        out_shape=jax.ShapeDtypeStruct((1, 128), jnp.int32),
    )(t1)


def _derive(k, t1, t2):
    return pl.pallas_call(
        functools.partial(_derive_kernel, k),
        out_specs=[pl.BlockSpec(memory_space=pltpu.SMEM)] * 3,
        out_shape=[jax.ShapeDtypeStruct((1, 1), jnp.int32)] * 3,
    )(t1, t2)


def _cutoff(pn, t, need, c2, n_chunks=16):
    n_pre, n_post = pn.shape

    def all_ties(ops):
        return jnp.full((1, 1), 1 << 24, jnp.int32)

    def exact_ties(ops):
        pn_, t_, need_ = ops
        return pl.pallas_call(
            functools.partial(_tie_kernel, n_chunks),
            in_specs=[
                pl.BlockSpec((n_pre, n_post), lambda: (0, 0)),
                pl.BlockSpec(memory_space=pltpu.SMEM),
                pl.BlockSpec(memory_space=pltpu.SMEM),
            ],
            out_specs=pl.BlockSpec(memory_space=pltpu.SMEM),
            out_shape=jax.ShapeDtypeStruct((1, 1), jnp.int32),
        )(pn_, t_, need_)

    return jax.lax.cond(c2[0, 0] == need[0, 0], all_ties, exact_ties,
                        (pn, t, need))


def _mask(pn, t, cut, row_block=256):
    n_pre, n_post = pn.shape
    nr = n_pre // row_block
    return pl.pallas_call(
        functools.partial(_mask_kernel, row_block),
        grid=(nr,),
        in_specs=[
            pl.BlockSpec(memory_space=pltpu.SMEM),
            pl.BlockSpec(memory_space=pltpu.SMEM),
            pl.BlockSpec((row_block, n_post), lambda i: (i, 0)),
        ],
        out_specs=pl.BlockSpec((row_block, n_post), lambda i: (i, 0)),
        out_shape=jax.ShapeDtypeStruct((n_pre, n_post), jnp.int32),
    )(t, cut, pn)


def kernel(x, h, y, perm_xy, perm_xh, perm_hy):
    triples = ((x, y, perm_xy), (x, h, perm_xh), (h, y, perm_hy))
    pns, pfs, ks = [], [], []
    for pre, post, perm in triples:
        pn = _normalize(pre, post, perm)
        pns.append(pn)
        pfs.append(jax.lax.bitcast_convert_type(pn.reshape(-1), jnp.int32))
        ks.append(math.ceil(perm.shape[0] * perm.shape[1] * _SPARSITY))
    t1s = [_make_sc_hist1(pf.shape[0])(pf) for pf in pfs]
    b1s = [_find_b1(k, t1) for k, t1 in zip(ks, t1s)]
    t2s = [_make_sc_hist2(pf.shape[0])(pf, b1)
           for pf, b1 in zip(pfs, b1s)]
    masks = []
    for k, pn, t1, t2 in zip(ks, pns, t1s, t2s):
        t, need, c2 = _derive(k, t1, t2)
        cut = _cutoff(pn, t, need, c2)
        masks.append(_mask(pn, t, cut))
    return (masks[0], masks[1], masks[2], pns[0], pns[1], pns[2])


# speedup vs baseline: 2.7594x; 2.0652x over previous
import functools
import math

import jax
import jax.numpy as jnp
from jax import lax
from jax.experimental import pallas as pl
from jax.experimental.pallas import tpu as pltpu
from jax.experimental.pallas import tpu_sc as plsc

_ALPHA = 0.001
_SPARSITY = 0.05

_NC = 2
_NS = 16
_NW = _NC * _NS
_LANES = 16
_NBINS = 1 << 15
_SUB = 16384
_BPT = _NBINS // _NS
_UNROLL = 8


def _normalize_kernel(pre_ref, post_ref, perm_ref, out_ref):
    prod = jax.lax.dot_general(
        pre_ref[...], post_ref[...], (((0,), (0,)), ((), ())),
        preferred_element_type=jnp.float32)
    p = perm_ref[...] + _ALPHA * prod
    s = jnp.sum(p, axis=0, keepdims=True)
    out_ref[...] = p / s


def _sc_hist_body(pf_hbm, chunk, d_refs, sems, table_ref, level, b1):
    cid = lax.axis_index("c")
    sid = lax.axis_index("s")
    wid = cid * _NS + sid
    base = wid * chunk
    nsub = chunk // _SUB

    @plsc.parallel_loop(0, _NBINS // _LANES, unroll=_UNROLL)
    def _(i):
        table_ref[pl.ds(i * _LANES, _LANES)] = jnp.zeros((_LANES,), jnp.int32)

    ones = jnp.ones((_LANES,), jnp.int32)

    def start(s, bref, sem):
        off = base + jnp.minimum(s, nsub - 1) * _SUB
        pltpu.async_copy(pf_hbm.at[pl.ds(off, _SUB)], bref, sem)

    def wait(bref, sem):
        pltpu.make_async_copy(pf_hbm.at[pl.ds(base, _SUB)], bref, sem).wait()

    def process(bref):
        @plsc.parallel_loop(0, _SUB // _LANES, unroll=_UNROLL)
        def _(i):
            u = bref[pl.ds(i * _LANES, _LANES)]
            if level == 0:
                b = jax.lax.shift_right_logical(u, 15)
                plsc.addupdate_scatter(table_ref, [b], ones)
            else:
                hi = jax.lax.shift_right_logical(u, 15)
                b = jnp.bitwise_and(u, jnp.int32(_NBINS - 1))
                plsc.addupdate_scatter(table_ref, [b], ones,
                                       mask=(hi == b1))

    start(0, d_refs[0], sems[0])
    start(1, d_refs[1], sems[1])

    def pair_body(p, _):
        s = p * 2
        wait(d_refs[0], sems[0])
        process(d_refs[0])
        start(s + 2, d_refs[0], sems[0])
        wait(d_refs[1], sems[1])
        process(d_refs[1])
        start(s + 3, d_refs[1], sems[1])
        return 0
    lax.fori_loop(0, nsub // 2, pair_body, 0)
    wait(d_refs[0], sems[0])
    wait(d_refs[1], sems[1])


_SLICE = 512


def _sc_merge_and_emit(out_hbm, table_ref, shared_ref, slice_ref):
    cid = lax.axis_index("c")
    sid = lax.axis_index("s")
    pltpu.sync_copy(table_ref, shared_ref.at[sid])
    plsc.subcore_barrier()
    for piece in range(_BPT // _SLICE):
        pltpu.sync_copy(
            shared_ref.at[:, pl.ds(sid * _BPT + piece * _SLICE, _SLICE)],
            slice_ref)

        @plsc.parallel_loop(0, _SLICE // _LANES, unroll=4)
        def _(i):
            acc = jnp.zeros((_LANES,), jnp.int32)
            for r in range(_NS):
                acc = acc + slice_ref[r, pl.ds(i * _LANES, _LANES)]
            table_ref[pl.ds(piece * _SLICE + i * _LANES, _LANES)] = acc
    pltpu.sync_copy(table_ref.at[pl.ds(0, _BPT)],
                    out_hbm.at[cid, pl.ds(sid * _BPT, _BPT)])


def _sc_mesh():
    return plsc.VectorSubcoreMesh(
        core_axis_name="c", subcore_axis_name="s",
        num_cores=_NC, num_subcores=_NS)


def _make_sc_hist1(n):
    chunk = n // _NW

    @functools.partial(
        pl.kernel,
        out_type=jax.ShapeDtypeStruct((_NC, _NBINS), jnp.int32),
        mesh=_sc_mesh(),
        compiler_params=pltpu.CompilerParams(needs_layout_passes=False),
        scratch_types=[
            pltpu.VMEM((_SUB,), jnp.int32),
            pltpu.VMEM((_SUB,), jnp.int32),
            pltpu.VMEM((_NBINS,), jnp.int32),
            pltpu.VMEM_SHARED((_NS, _NBINS), jnp.int32),
            pltpu.VMEM((_NS, _SLICE), jnp.int32),
            pltpu.SemaphoreType.DMA,
            pltpu.SemaphoreType.DMA,
        ],
    )
    def hist1(pf_hbm, out_hbm, d0, d1, table_ref, shared_ref, slice_ref,
              sem0, sem1):
        _sc_hist_body(pf_hbm, chunk, (d0, d1), (sem0, sem1), table_ref,
                      0, None)
        _sc_merge_and_emit(out_hbm, table_ref, shared_ref, slice_ref)

    return hist1


def _make_sc_hist2(n):
    chunk = n // _NW

    @functools.partial(
        pl.kernel,
        out_type=jax.ShapeDtypeStruct((_NC, _NBINS), jnp.int32),
        mesh=_sc_mesh(),
        compiler_params=pltpu.CompilerParams(needs_layout_passes=False),
        scratch_types=[
            pltpu.VMEM((_SUB,), jnp.int32),
            pltpu.VMEM((_SUB,), jnp.int32),
            pltpu.VMEM((_NBINS,), jnp.int32),
            pltpu.VMEM_SHARED((_NS, _NBINS), jnp.int32),
            pltpu.VMEM((_NS, _SLICE), jnp.int32),
            pltpu.VMEM((_LANES,), jnp.int32),
            pltpu.SemaphoreType.DMA,
            pltpu.SemaphoreType.DMA,
        ],
    )
    def hist2(pf_hbm, b1_hbm, out_hbm, d0, d1, table_ref, shared_ref,
              slice_ref, b1_ref, sem0, sem1):
        pltpu.sync_copy(b1_hbm.at[0, pl.ds(0, _LANES)], b1_ref)
        v = b1_ref[...]
        lanes = lax.iota(jnp.int32, _LANES)
        b1 = jnp.sum(jnp.where(lanes == 0, v, jnp.zeros((_LANES,), jnp.int32)))
        _sc_hist_body(pf_hbm, chunk, (d0, d1), (sem0, sem1), table_ref,
                      1, b1)
        _sc_merge_and_emit(out_hbm, table_ref, shared_ref, slice_ref)

    return hist2


def _suffix_find(m, bins, kk):
    def bit_body(i, b):
        trial = jnp.bitwise_or(b, jax.lax.shift_left(jnp.int32(1), jnp.int32(14) - i))
        s = jnp.sum(jnp.where(bins >= trial, m, 0))
        return jnp.where(s >= kk, trial, b)
    return jax.lax.fori_loop(0, 15, bit_body, jnp.int32(0))


def _find_b1_kernel(k, t1_ref, out_ref):
    bins = jax.lax.broadcasted_iota(jnp.int32, (1, _NBINS), 1)
    m1 = jnp.sum(t1_ref[...], axis=0, keepdims=True)
    b1 = _suffix_find(m1, bins, k)
    lanes = jax.lax.broadcasted_iota(jnp.int32, (1, 128), 1)
    out_ref[...] = jnp.where(lanes == 0, b1, 0)


def _derive_kernel(k, t1_ref, t2_ref, t_ref, need_ref, c2_ref):
    bins = jax.lax.broadcasted_iota(jnp.int32, (1, _NBINS), 1)
    m1 = jnp.sum(t1_ref[...], axis=0, keepdims=True)
    b1 = _suffix_find(m1, bins, k)
    krem = k - jnp.sum(jnp.where(bins > b1, m1, 0))
    m2 = jnp.sum(t2_ref[...], axis=0, keepdims=True)
    b2 = _suffix_find(m2, bins, krem)
    s2_excl = jnp.sum(jnp.where(bins > b2, m2, 0))
    c2 = jnp.sum(jnp.where(bins == b2, m2, 0))
    t_ref[0, 0] = jnp.bitwise_or(jax.lax.shift_left(b1, 15), b2)
    need_ref[0, 0] = krem - s2_excl
    c2_ref[0, 0] = c2


def _tie_kernel(n_chunks, pn_ref, t_ref, need_ref, c_ref):
    n_rows, n_cols = pn_ref.shape
    r = n_rows // n_chunks
    t = t_ref[0, 0]
    need = need_ref[0, 0]

    def count_eq_lt(cut):
        def body(ci, acc):
            sl = pl.ds(pl.multiple_of(ci * r, r), r)
            u = jax.lax.bitcast_convert_type(pn_ref[sl, :], jnp.int32)
            rows = jax.lax.broadcasted_iota(jnp.int32, (r, n_cols), 0)
            cols = jax.lax.broadcasted_iota(jnp.int32, (r, n_cols), 1)
            flat = (rows + ci * r) * n_cols + cols
            return acc + jnp.sum(((u == t) & (flat < cut)).astype(jnp.int32))
        return jax.lax.fori_loop(0, n_chunks, body, jnp.int32(0))

    def c_body(i, c):
        trial = jnp.bitwise_or(c, jax.lax.shift_left(jnp.int32(1), jnp.int32(23) - i))
        return jnp.where(count_eq_lt(trial) <= need, trial, c)

    c_ref[0, 0] = jax.lax.fori_loop(0, 24, c_body, jnp.int32(0))


def _mask_kernel(row_block, t_ref, c_ref, pn_ref, mask_ref):
    u = jax.lax.bitcast_convert_type(pn_ref[...], jnp.int32)
    t = t_ref[0, 0]
    cut = c_ref[0, 0]
    n_cols = u.shape[1]
    rows = jax.lax.broadcasted_iota(jnp.int32, u.shape, 0)
    cols = jax.lax.broadcasted_iota(jnp.int32, u.shape, 1)
    flat = (rows + pl.program_id(0) * row_block) * n_cols + cols
    mask_ref[...] = ((u > t) | ((u == t) & (flat < cut))).astype(jnp.int32)


def _normalize(pre, post, perm, col_block=512):
    n_pre, n_post = perm.shape
    b = pre.shape[0]
    nc = n_post // col_block
    return pl.pallas_call(
        _normalize_kernel,
        grid=(nc,),
        in_specs=[
            pl.BlockSpec((b, n_pre), lambda j: (0, 0)),
            pl.BlockSpec((b, col_block), lambda j: (0, j)),
            pl.BlockSpec((n_pre, col_block), lambda j: (0, j)),
        ],
        out_specs=pl.BlockSpec((n_pre, col_block), lambda j: (0, j)),
        out_shape=jax.ShapeDtypeStruct((n_pre, n_post), jnp.float32),
    )(pre, post, perm)


def _find_b1(k, t1):
    return pl.pallas_call(
        functools.partial(_find_b1_kernel, k),
        out_shape=jax.ShapeDtypeStruct((1, 128), jnp.int32),
    )(t1)


def _derive(k, t1, t2):
    return pl.pallas_call(
        functools.partial(_derive_kernel, k),
        out_specs=[pl.BlockSpec(memory_space=pltpu.SMEM)] * 3,
        out_shape=[jax.ShapeDtypeStruct((1, 1), jnp.int32)] * 3,
    )(t1, t2)


def _cutoff(pn, t, need, c2, n_chunks=16):
    n_pre, n_post = pn.shape

    def all_ties(ops):
        return jnp.full((1, 1), 1 << 24, jnp.int32)

    def exact_ties(ops):
        pn_, t_, need_ = ops
        return pl.pallas_call(
            functools.partial(_tie_kernel, n_chunks),
            in_specs=[
                pl.BlockSpec((n_pre, n_post), lambda: (0, 0)),
                pl.BlockSpec(memory_space=pltpu.SMEM),
                pl.BlockSpec(memory_space=pltpu.SMEM),
            ],
            out_specs=pl.BlockSpec(memory_space=pltpu.SMEM),
            out_shape=jax.ShapeDtypeStruct((1, 1), jnp.int32),
        )(pn_, t_, need_)

    return jax.lax.cond(c2[0, 0] == need[0, 0], all_ties, exact_ties,
                        (pn, t, need))


def _mask(pn, t, cut, row_block=256):
    n_pre, n_post = pn.shape
    nr = n_pre // row_block
    return pl.pallas_call(
        functools.partial(_mask_kernel, row_block),
        grid=(nr,),
        in_specs=[
            pl.BlockSpec(memory_space=pltpu.SMEM),
            pl.BlockSpec(memory_space=pltpu.SMEM),
            pl.BlockSpec((row_block, n_post), lambda i: (i, 0)),
        ],
        out_specs=pl.BlockSpec((row_block, n_post), lambda i: (i, 0)),
        out_shape=jax.ShapeDtypeStruct((n_pre, n_post), jnp.int32),
    )(t, cut, pn)


def kernel(x, h, y, perm_xy, perm_xh, perm_hy):
    triples = ((x, y, perm_xy), (x, h, perm_xh), (h, y, perm_hy))
    pns, pfs, ks = [], [], []
    for pre, post, perm in triples:
        pn = _normalize(pre, post, perm)
        pns.append(pn)
        pfs.append(jax.lax.bitcast_convert_type(pn.reshape(-1), jnp.int32))
        ks.append(math.ceil(perm.shape[0] * perm.shape[1] * _SPARSITY))
    t1s = [_make_sc_hist1(pf.shape[0])(pf) for pf in pfs]
    b1s = [_find_b1(k, t1) for k, t1 in zip(ks, t1s)]
    t2s = [_make_sc_hist2(pf.shape[0])(pf, b1)
           for pf, b1 in zip(pfs, b1s)]
    masks = []
    for k, pn, t1, t2 in zip(ks, pns, t1s, t2s):
        t, need, c2 = _derive(k, t1, t2)
        cut = _cutoff(pn, t, need, c2)
        masks.append(_mask(pn, t, cut))
    return (masks[0], masks[1], masks[2], pns[0], pns[1], pns[2])


# speedup vs baseline: 3.1388x; 1.1375x over previous
import functools
import math

import jax
import jax.numpy as jnp
from jax import lax
from jax.experimental import pallas as pl
from jax.experimental.pallas import tpu as pltpu
from jax.experimental.pallas import tpu_sc as plsc

_ALPHA = 0.001
_SPARSITY = 0.05

_NC = 2
_NS = 16
_NW = _NC * _NS
_LANES = 16
_NBINS = 1 << 15
_SUB = 16384
_BPT = _NBINS // _NS
_UNROLL = 8


def _normalize_kernel(pre_ref, post_ref, perm_ref, out_ref):
    prod = jax.lax.dot_general(
        pre_ref[...], post_ref[...], (((0,), (0,)), ((), ())),
        preferred_element_type=jnp.float32)
    p = perm_ref[...] + _ALPHA * prod
    s = jnp.sum(p, axis=0, keepdims=True)
    out_ref[...] = p / s


def _sc_hist_body(pf_hbm, chunk, d_refs, sems, table_ref, level, b1):
    cid = lax.axis_index("c")
    sid = lax.axis_index("s")
    wid = cid * _NS + sid
    base = wid * chunk
    nsub = chunk // _SUB

    @plsc.parallel_loop(0, _NBINS // _LANES, unroll=_UNROLL)
    def _(i):
        table_ref[pl.ds(i * _LANES, _LANES)] = jnp.zeros((_LANES,), jnp.int32)

    ones = jnp.ones((_LANES,), jnp.int32)

    def start(s, bref, sem):
        off = base + jnp.minimum(s, nsub - 1) * _SUB
        pltpu.async_copy(pf_hbm.at[pl.ds(off, _SUB)], bref, sem)

    def wait(bref, sem):
        pltpu.make_async_copy(pf_hbm.at[pl.ds(base, _SUB)], bref, sem).wait()

    def process(bref):
        @plsc.parallel_loop(0, _SUB // _LANES, unroll=_UNROLL)
        def _(i):
            u = plsc.bitcast(bref[pl.ds(i * _LANES, _LANES)], jnp.int32)
            if level == 0:
                b = jax.lax.shift_right_logical(u, 15)
                plsc.addupdate_scatter(table_ref, [b], ones)
            else:
                hi = jax.lax.shift_right_logical(u, 15)
                b = jnp.bitwise_and(u, jnp.int32(_NBINS - 1))
                plsc.addupdate_scatter(table_ref, [b], ones,
                                       mask=(hi == b1))

    start(0, d_refs[0], sems[0])
    start(1, d_refs[1], sems[1])

    def pair_body(p, _):
        s = p * 2
        wait(d_refs[0], sems[0])
        process(d_refs[0])
        start(s + 2, d_refs[0], sems[0])
        wait(d_refs[1], sems[1])
        process(d_refs[1])
        start(s + 3, d_refs[1], sems[1])
        return 0
    lax.fori_loop(0, nsub // 2, pair_body, 0)
    wait(d_refs[0], sems[0])
    wait(d_refs[1], sems[1])


_SLICE = 512


def _sc_merge_and_emit(out_hbm, table_ref, shared_ref, slice_ref):
    cid = lax.axis_index("c")
    sid = lax.axis_index("s")
    pltpu.sync_copy(table_ref, shared_ref.at[sid])
    plsc.subcore_barrier()
    for piece in range(_BPT // _SLICE):
        pltpu.sync_copy(
            shared_ref.at[:, pl.ds(sid * _BPT + piece * _SLICE, _SLICE)],
            slice_ref)

        @plsc.parallel_loop(0, _SLICE // _LANES, unroll=4)
        def _(i):
            acc = jnp.zeros((_LANES,), jnp.int32)
            for r in range(_NS):
                acc = acc + slice_ref[r, pl.ds(i * _LANES, _LANES)]
            table_ref[pl.ds(piece * _SLICE + i * _LANES, _LANES)] = acc
    pltpu.sync_copy(table_ref.at[pl.ds(0, _BPT)],
                    out_hbm.at[cid, pl.ds(sid * _BPT, _BPT)])


def _sc_mesh():
    return plsc.VectorSubcoreMesh(
        core_axis_name="c", subcore_axis_name="s",
        num_cores=_NC, num_subcores=_NS)


def _make_sc_hist1(n):
    chunk = n // _NW

    @functools.partial(
        pl.kernel,
        out_type=jax.ShapeDtypeStruct((_NC, _NBINS), jnp.int32),
        mesh=_sc_mesh(),
        compiler_params=pltpu.CompilerParams(needs_layout_passes=False),
        scratch_types=[
            pltpu.VMEM((_SUB,), jnp.float32),
            pltpu.VMEM((_SUB,), jnp.float32),
            pltpu.VMEM((_NBINS,), jnp.int32),
            pltpu.VMEM_SHARED((_NS, _NBINS), jnp.int32),
            pltpu.VMEM((_NS, _SLICE), jnp.int32),
            pltpu.SemaphoreType.DMA,
            pltpu.SemaphoreType.DMA,
        ],
    )
    def hist1(pf_hbm, out_hbm, d0, d1, table_ref, shared_ref, slice_ref,
              sem0, sem1):
        _sc_hist_body(pf_hbm, chunk, (d0, d1), (sem0, sem1), table_ref,
                      0, None)
        _sc_merge_and_emit(out_hbm, table_ref, shared_ref, slice_ref)

    return hist1


def _make_sc_hist2(n):
    chunk = n // _NW

    @functools.partial(
        pl.kernel,
        out_type=jax.ShapeDtypeStruct((_NC, _NBINS), jnp.int32),
        mesh=_sc_mesh(),
        compiler_params=pltpu.CompilerParams(needs_layout_passes=False),
        scratch_types=[
            pltpu.VMEM((_SUB,), jnp.float32),
            pltpu.VMEM((_SUB,), jnp.float32),
            pltpu.VMEM((_NBINS,), jnp.int32),
            pltpu.VMEM_SHARED((_NS, _NBINS), jnp.int32),
            pltpu.VMEM((_NS, _SLICE), jnp.int32),
            pltpu.VMEM((_LANES,), jnp.int32),
            pltpu.SemaphoreType.DMA,
            pltpu.SemaphoreType.DMA,
        ],
    )
    def hist2(pf_hbm, b1_hbm, out_hbm, d0, d1, table_ref, shared_ref,
              slice_ref, b1_ref, sem0, sem1):
        pltpu.sync_copy(b1_hbm.at[0, pl.ds(0, _LANES)], b1_ref)
        v = b1_ref[...]
        lanes = lax.iota(jnp.int32, _LANES)
        b1 = jnp.sum(jnp.where(lanes == 0, v, jnp.zeros((_LANES,), jnp.int32)))
        _sc_hist_body(pf_hbm, chunk, (d0, d1), (sem0, sem1), table_ref,
                      1, b1)
        _sc_merge_and_emit(out_hbm, table_ref, shared_ref, slice_ref)

    return hist2


def _suffix_find(m, bins, kk):
    def bit_body(i, b):
        trial = jnp.bitwise_or(b, jax.lax.shift_left(jnp.int32(1), jnp.int32(14) - i))
        s = jnp.sum(jnp.where(bins >= trial, m, 0))
        return jnp.where(s >= kk, trial, b)
    return jax.lax.fori_loop(0, 15, bit_body, jnp.int32(0))


def _find_b1_kernel(k, t1_ref, out_ref):
    bins = jax.lax.broadcasted_iota(jnp.int32, (1, _NBINS), 1)
    m1 = jnp.sum(t1_ref[...], axis=0, keepdims=True)
    b1 = _suffix_find(m1, bins, k)
    lanes = jax.lax.broadcasted_iota(jnp.int32, (1, 128), 1)
    out_ref[...] = jnp.where(lanes == 0, b1, 0)


def _derive_kernel(k, t1_ref, t2_ref, t_ref, need_ref, c2_ref):
    bins = jax.lax.broadcasted_iota(jnp.int32, (1, _NBINS), 1)
    m1 = jnp.sum(t1_ref[...], axis=0, keepdims=True)
    b1 = _suffix_find(m1, bins, k)
    krem = k - jnp.sum(jnp.where(bins > b1, m1, 0))
    m2 = jnp.sum(t2_ref[...], axis=0, keepdims=True)
    b2 = _suffix_find(m2, bins, krem)
    s2_excl = jnp.sum(jnp.where(bins > b2, m2, 0))
    c2 = jnp.sum(jnp.where(bins == b2, m2, 0))
    t_ref[0, 0] = jnp.bitwise_or(jax.lax.shift_left(b1, 15), b2)
    need_ref[0, 0] = krem - s2_excl
    c2_ref[0, 0] = c2


def _tie_kernel(n_chunks, pn_ref, t_ref, need_ref, c_ref):
    n_rows, n_cols = pn_ref.shape
    r = n_rows // n_chunks
    t = t_ref[0, 0]
    need = need_ref[0, 0]

    def count_eq_lt(cut):
        def body(ci, acc):
            sl = pl.ds(pl.multiple_of(ci * r, r), r)
            u = jax.lax.bitcast_convert_type(pn_ref[sl, :], jnp.int32)
            rows = jax.lax.broadcasted_iota(jnp.int32, (r, n_cols), 0)
            cols = jax.lax.broadcasted_iota(jnp.int32, (r, n_cols), 1)
            flat = (rows + ci * r) * n_cols + cols
            return acc + jnp.sum(((u == t) & (flat < cut)).astype(jnp.int32))
        return jax.lax.fori_loop(0, n_chunks, body, jnp.int32(0))

    def c_body(i, c):
        trial = jnp.bitwise_or(c, jax.lax.shift_left(jnp.int32(1), jnp.int32(23) - i))
        return jnp.where(count_eq_lt(trial) <= need, trial, c)

    c_ref[0, 0] = jax.lax.fori_loop(0, 24, c_body, jnp.int32(0))


def _mask_kernel(row_block, t_ref, c_ref, pn_ref, mask_ref):
    u = jax.lax.bitcast_convert_type(pn_ref[...], jnp.int32)
    t = t_ref[0, 0]
    cut = c_ref[0, 0]
    n_cols = u.shape[1]
    rows = jax.lax.broadcasted_iota(jnp.int32, u.shape, 0)
    cols = jax.lax.broadcasted_iota(jnp.int32, u.shape, 1)
    flat = (rows + pl.program_id(0) * row_block) * n_cols + cols
    mask_ref[...] = ((u > t) | ((u == t) & (flat < cut))).astype(jnp.int32)


def _normalize(pre, post, perm, col_block=512):
    n_pre, n_post = perm.shape
    b = pre.shape[0]
    nc = n_post // col_block
    return pl.pallas_call(
        _normalize_kernel,
        grid=(nc,),
        in_specs=[
            pl.BlockSpec((b, n_pre), lambda j: (0, 0)),
            pl.BlockSpec((b, col_block), lambda j: (0, j)),
            pl.BlockSpec((n_pre, col_block), lambda j: (0, j)),
        ],
        out_specs=pl.BlockSpec((n_pre, col_block), lambda j: (0, j)),
        out_shape=jax.ShapeDtypeStruct((n_pre, n_post), jnp.float32),
    )(pre, post, perm)


def _find_b1(k, t1):
    return pl.pallas_call(
        functools.partial(_find_b1_kernel, k),
        out_shape=jax.ShapeDtypeStruct((1, 128), jnp.int32),
    )(t1)


def _derive(k, t1, t2):
    return pl.pallas_call(
        functools.partial(_derive_kernel, k),
        out_specs=[pl.BlockSpec(memory_space=pltpu.SMEM)] * 3,
        out_shape=[jax.ShapeDtypeStruct((1, 1), jnp.int32)] * 3,
    )(t1, t2)


def _cutoff(pn, t, need, c2, n_chunks=16):
    n_pre, n_post = pn.shape

    def all_ties(ops):
        return jnp.full((1, 1), 1 << 24, jnp.int32)

    def exact_ties(ops):
        pn_, t_, need_ = ops
        return pl.pallas_call(
            functools.partial(_tie_kernel, n_chunks),
            in_specs=[
                pl.BlockSpec((n_pre, n_post), lambda: (0, 0)),
                pl.BlockSpec(memory_space=pltpu.SMEM),
                pl.BlockSpec(memory_space=pltpu.SMEM),
            ],
            out_specs=pl.BlockSpec(memory_space=pltpu.SMEM),
            out_shape=jax.ShapeDtypeStruct((1, 1), jnp.int32),
        )(pn_, t_, need_)

    return jax.lax.cond(c2[0, 0] == need[0, 0], all_ties, exact_ties,
                        (pn, t, need))


def _mask(pn, t, cut, row_block=256):
    n_pre, n_post = pn.shape
    nr = n_pre // row_block
    return pl.pallas_call(
        functools.partial(_mask_kernel, row_block),
        grid=(nr,),
        in_specs=[
            pl.BlockSpec(memory_space=pltpu.SMEM),
            pl.BlockSpec(memory_space=pltpu.SMEM),
            pl.BlockSpec((row_block, n_post), lambda i: (i, 0)),
        ],
        out_specs=pl.BlockSpec((row_block, n_post), lambda i: (i, 0)),
        out_shape=jax.ShapeDtypeStruct((n_pre, n_post), jnp.int32),
    )(t, cut, pn)


def kernel(x, h, y, perm_xy, perm_xh, perm_hy):
    triples = ((x, y, perm_xy), (x, h, perm_xh), (h, y, perm_hy))
    pns, pfs, ks = [], [], []
    for pre, post, perm in triples:
        pn = _normalize(pre, post, perm)
        pns.append(pn)
        pfs.append(pn.reshape(-1))
        ks.append(math.ceil(perm.shape[0] * perm.shape[1] * _SPARSITY))
    t1s = [_make_sc_hist1(pf.shape[0])(pf) for pf in pfs]
    b1s = [_find_b1(k, t1) for k, t1 in zip(ks, t1s)]
    t2s = [_make_sc_hist2(pf.shape[0])(pf, b1)
           for pf, b1 in zip(pfs, b1s)]
    masks = []
    for k, pn, t1, t2 in zip(ks, pns, t1s, t2s):
        t, need, c2 = _derive(k, t1, t2)
        cut = _cutoff(pn, t, need, c2)
        masks.append(_mask(pn, t, cut))
    return (masks[0], masks[1], masks[2], pns[0], pns[1], pns[2])
